# baseline (speedup 1.0000x reference)
import jax
import jax.numpy as jnp
from jax.experimental import pallas as pl
from jax.experimental.pallas import tpu as pltpu

_T = 256


def _rvq_block(z_ref, cb_ref, quant_ref, idx_ref, loss_ref):
    i = pl.program_id(0)
    z = z_ref[...]
    cb = cb_ref[...]
    k = cb.shape[0]
    cb_sq = jnp.sum(cb * cb, axis=1)

    def stage(r):
        scores = jax.lax.dot_general(
            r.astype(jnp.bfloat16), cb.astype(jnp.bfloat16),
            (((1,), (1,)), ((), ())),
            preferred_element_type=jnp.float32)
        r_sq = jnp.sum(r * r, axis=1, keepdims=True)
        dist = r_sq - 2.0 * scores + cb_sq[None, :]
        minval = jnp.min(dist, axis=1, keepdims=True)
        iota = jax.lax.broadcasted_iota(jnp.int32, dist.shape, 1)
        idx = jnp.min(jnp.where(dist == minval, iota, k), axis=1)
        onehot = (iota == idx[:, None]).astype(jnp.float32)
        q = jnp.dot(onehot, cb, preferred_element_type=jnp.float32,
                    precision=jax.lax.Precision.HIGHEST)
        loss = jnp.sum((r - q) ** 2)
        return idx, q, loss

    idx1, q1, l1 = stage(z)
    r2 = z - q1
    idx2, q2, l2 = stage(r2)

    quant_ref[...] = q1 + q2
    idx_ref[...] = jnp.stack([idx1, idx2])[None]

    @pl.when(i == 0)
    def _():
        loss_ref[...] = jnp.zeros_like(loss_ref)

    loss_ref[...] += jnp.reshape(l1 + l2, (1, 1))


def kernel(z, codebook):
    b, s, d = z.shape
    k = codebook.shape[0]
    n_tok = b * s
    n_blk = n_tok // _T
    z_flat = z.reshape(n_tok, d)

    quant, idx, loss = pl.pallas_call(
        _rvq_block,
        grid=(n_blk,),
        in_specs=[
            pl.BlockSpec((_T, d), lambda i: (i, 0)),
            pl.BlockSpec((k, d), lambda i: (0, 0)),
        ],
        out_specs=[
            pl.BlockSpec((_T, d), lambda i: (i, 0)),
            pl.BlockSpec((1, 2, _T), lambda i: (i, 0, 0)),
            pl.BlockSpec((1, 1), lambda i: (0, 0)),
        ],
        out_shape=[
            jax.ShapeDtypeStruct((n_tok, d), jnp.float32),
            jax.ShapeDtypeStruct((n_blk, 2, _T), jnp.int32),
            jax.ShapeDtypeStruct((1, 1), jnp.float32),
        ],
        compiler_params=pltpu.CompilerParams(
            dimension_semantics=("arbitrary",),
        ),
    )(z_flat, codebook)

    quant_out = quant.reshape(b, s, d)
    codebook_indices = idx.transpose(0, 2, 1).reshape(b, s, 2)
    q_loss = loss[0, 0] * jnp.float32(2.0 / (n_tok * d))
    return quant_out, codebook_indices, q_loss


# speedup vs baseline: 2.5753x; 2.5753x over previous
import jax
import jax.numpy as jnp
from jax.experimental import pallas as pl
from jax.experimental.pallas import tpu as pltpu

_T = 256


def _rvq_block(z_ref, cb_ref, quant_ref, idx_ref, loss_ref):
    i = pl.program_id(0)
    z = z_ref[...]
    cb = cb_ref[...]
    cb_sq = jnp.sum(cb * cb, axis=1)
    cb_b16 = cb.astype(jnp.bfloat16)
    cb_hi = cb_b16.astype(jnp.float32)
    rem = cb - cb_hi
    cb_mid = rem.astype(jnp.bfloat16).astype(jnp.float32)
    cb_lo = rem - cb_mid

    def stage(r):
        scores = jax.lax.dot_general(
            r.astype(jnp.bfloat16), cb_b16,
            (((1,), (1,)), ((), ())),
            preferred_element_type=jnp.float32)
        r_sq = jnp.sum(r * r, axis=1, keepdims=True)
        dist = r_sq - 2.0 * scores + cb_sq[None, :]
        idx = jnp.argmin(dist, axis=1).astype(jnp.int32)
        iota = jax.lax.broadcasted_iota(jnp.int32, dist.shape, 1)
        onehot = (iota == idx[:, None]).astype(jnp.bfloat16)
        q = (jnp.dot(onehot, cb_mid.astype(jnp.bfloat16),
                     preferred_element_type=jnp.float32)
             + jnp.dot(onehot, cb_lo.astype(jnp.bfloat16),
                       preferred_element_type=jnp.float32)
             + jnp.dot(onehot, cb_b16,
                       preferred_element_type=jnp.float32))
        loss = jnp.sum((r - q) ** 2)
        return idx, q, loss

    idx1, q1, l1 = stage(z)
    r2 = z - q1
    idx2, q2, l2 = stage(r2)

    quant_ref[...] = q1 + q2
    idx_ref[...] = jnp.stack([idx1, idx2])[None]

    @pl.when(i == 0)
    def _():
        loss_ref[...] = jnp.zeros_like(loss_ref)

    loss_ref[...] += jnp.reshape(l1 + l2, (1, 1))


def kernel(z, codebook):
    b, s, d = z.shape
    k = codebook.shape[0]
    n_tok = b * s
    n_blk = n_tok // _T
    z_flat = z.reshape(n_tok, d)

    quant, idx, loss = pl.pallas_call(
        _rvq_block,
        grid=(n_blk,),
        in_specs=[
            pl.BlockSpec((_T, d), lambda i: (i, 0)),
            pl.BlockSpec((k, d), lambda i: (0, 0)),
        ],
        out_specs=[
            pl.BlockSpec((_T, d), lambda i: (i, 0)),
            pl.BlockSpec((1, 2, _T), lambda i: (i, 0, 0)),
            pl.BlockSpec((1, 1), lambda i: (0, 0)),
        ],
        out_shape=[
            jax.ShapeDtypeStruct((n_tok, d), jnp.float32),
            jax.ShapeDtypeStruct((n_blk, 2, _T), jnp.int32),
            jax.ShapeDtypeStruct((1, 1), jnp.float32),
        ],
        compiler_params=pltpu.CompilerParams(
            dimension_semantics=("arbitrary",),
        ),
    )(z_flat, codebook)

    quant_out = quant.reshape(b, s, d)
    codebook_indices = idx.transpose(0, 2, 1).reshape(b, s, 2)
    q_loss = loss[0, 0] * jnp.float32(2.0 / (n_tok * d))
    return quant_out, codebook_indices, q_loss


# speedup vs baseline: 3.2882x; 1.2768x over previous
import jax
import jax.numpy as jnp
from jax.experimental import pallas as pl
from jax.experimental.pallas import tpu as pltpu

_T = 256


def _rvq_block(z_ref, cb_ref, quant_ref, idx_ref, loss_ref,
               cb16_ref, w_ref, cbsq_ref):
    i = pl.program_id(0)

    @pl.when(i == 0)
    def _():
        cb = cb_ref[...]
        cb_b16 = cb.astype(jnp.bfloat16)
        cb_hi = cb_b16.astype(jnp.float32)
        rem = cb - cb_hi
        cb_mid = rem.astype(jnp.bfloat16)
        cb_lo = rem - cb_mid.astype(jnp.float32)
        cb16_ref[...] = cb_b16
        w_ref[...] = jnp.concatenate(
            [cb_b16, cb_mid, cb_lo.astype(jnp.bfloat16)], axis=1)
        cbsq_ref[...] = jnp.sum(cb * cb, axis=1)[None, :]
        loss_ref[...] = jnp.zeros_like(loss_ref)

    z = z_ref[...]
    cb16 = cb16_ref[...]
    w = w_ref[...]
    cb_sq = cbsq_ref[...]
    d = z.shape[1]

    def stage(r):
        scores = jax.lax.dot_general(
            r.astype(jnp.bfloat16), cb16,
            (((1,), (1,)), ((), ())),
            preferred_element_type=jnp.float32)
        r_sq = jnp.sum(r * r, axis=1, keepdims=True)
        dist = r_sq - 2.0 * scores + cb_sq
        idx = jnp.argmin(dist, axis=1).astype(jnp.int32)
        iota = jax.lax.broadcasted_iota(jnp.int32, dist.shape, 1)
        onehot = (iota == idx[:, None]).astype(jnp.bfloat16)
        g = jnp.dot(onehot, w, preferred_element_type=jnp.float32)
        q = (g[:, d:2 * d] + g[:, 2 * d:]) + g[:, :d]
        loss = jnp.sum((r - q) ** 2)
        return idx, q, loss

    idx1, q1, l1 = stage(z)
    r2 = z - q1
    idx2, q2, l2 = stage(r2)

    quant_ref[...] = q1 + q2
    idx_ref[...] = jnp.stack([idx1, idx2])[None]
    loss_ref[...] += jnp.reshape(l1 + l2, (1, 1))


def kernel(z, codebook):
    b, s, d = z.shape
    k = codebook.shape[0]
    n_tok = b * s
    n_blk = n_tok // _T
    z_flat = z.reshape(n_tok, d)

    quant, idx, loss = pl.pallas_call(
        _rvq_block,
        grid=(n_blk,),
        in_specs=[
            pl.BlockSpec((_T, d), lambda i: (i, 0)),
            pl.BlockSpec((k, d), lambda i: (0, 0)),
        ],
        out_specs=[
            pl.BlockSpec((_T, d), lambda i: (i, 0)),
            pl.BlockSpec((1, 2, _T), lambda i: (i, 0, 0)),
            pl.BlockSpec((1, 1), lambda i: (0, 0)),
        ],
        out_shape=[
            jax.ShapeDtypeStruct((n_tok, d), jnp.float32),
            jax.ShapeDtypeStruct((n_blk, 2, _T), jnp.int32),
            jax.ShapeDtypeStruct((1, 1), jnp.float32),
        ],
        scratch_shapes=[
            pltpu.VMEM((k, d), jnp.bfloat16),
            pltpu.VMEM((k, 3 * d), jnp.bfloat16),
            pltpu.VMEM((1, k), jnp.float32),
        ],
        compiler_params=pltpu.CompilerParams(
            dimension_semantics=("arbitrary",),
        ),
    )(z_flat, codebook)

    quant_out = quant.reshape(b, s, d)
    codebook_indices = idx.transpose(0, 2, 1).reshape(b, s, 2)
    q_loss = loss[0, 0] * jnp.float32(2.0 / (n_tok * d))
    return quant_out, codebook_indices, q_loss


# speedup vs baseline: 3.3905x; 1.0311x over previous
import jax
import jax.numpy as jnp
from jax.experimental import pallas as pl
from jax.experimental.pallas import tpu as pltpu

_T = 512


def _rvq_block(z_ref, cb_ref, quant_ref, idx_ref, loss_ref,
               cb16_ref, w_ref, cbsq_ref):
    i = pl.program_id(0)

    @pl.when(i == 0)
    def _():
        cb = cb_ref[...]
        cb_b16 = cb.astype(jnp.bfloat16)
        cb_hi = cb_b16.astype(jnp.float32)
        rem = cb - cb_hi
        cb_mid = rem.astype(jnp.bfloat16)
        cb_lo = rem - cb_mid.astype(jnp.float32)
        cb16_ref[...] = cb_b16
        w_ref[...] = jnp.concatenate(
            [cb_b16, cb_mid, cb_lo.astype(jnp.bfloat16)], axis=1)
        cbsq_ref[...] = jnp.sum(cb * cb, axis=1)[None, :]
        loss_ref[...] = jnp.zeros_like(loss_ref)

    z = z_ref[...]
    cb16 = cb16_ref[...]
    w = w_ref[...]
    cb_sq = cbsq_ref[...]
    d = z.shape[1]

    def stage(r):
        scores = jax.lax.dot_general(
            r.astype(jnp.bfloat16), cb16,
            (((1,), (1,)), ((), ())),
            preferred_element_type=jnp.float32)
        r_sq = jnp.sum(r * r, axis=1, keepdims=True)
        dist = r_sq - 2.0 * scores + cb_sq
        idx = jnp.argmin(dist, axis=1).astype(jnp.int32)
        iota = jax.lax.broadcasted_iota(jnp.int32, dist.shape, 1)
        onehot = (iota == idx[:, None]).astype(jnp.bfloat16)
        g = jnp.dot(onehot, w, preferred_element_type=jnp.float32)
        q = (g[:, d:2 * d] + g[:, 2 * d:]) + g[:, :d]
        loss = jnp.sum((r - q) ** 2)
        return idx, q, loss

    idx1, q1, l1 = stage(z)
    r2 = z - q1
    idx2, q2, l2 = stage(r2)

    quant_ref[...] = q1 + q2
    idx_ref[...] = jnp.stack([idx1, idx2])[None]
    loss_ref[...] += jnp.reshape(l1 + l2, (1, 1))


def kernel(z, codebook):
    b, s, d = z.shape
    k = codebook.shape[0]
    n_tok = b * s
    n_blk = n_tok // _T
    z_flat = z.reshape(n_tok, d)

    quant, idx, loss = pl.pallas_call(
        _rvq_block,
        grid=(n_blk,),
        in_specs=[
            pl.BlockSpec((_T, d), lambda i: (i, 0)),
            pl.BlockSpec((k, d), lambda i: (0, 0)),
        ],
        out_specs=[
            pl.BlockSpec((_T, d), lambda i: (i, 0)),
            pl.BlockSpec((1, 2, _T), lambda i: (i, 0, 0)),
            pl.BlockSpec((1, 1), lambda i: (0, 0)),
        ],
        out_shape=[
            jax.ShapeDtypeStruct((n_tok, d), jnp.float32),
            jax.ShapeDtypeStruct((n_blk, 2, _T), jnp.int32),
            jax.ShapeDtypeStruct((1, 1), jnp.float32),
        ],
        scratch_shapes=[
            pltpu.VMEM((k, d), jnp.bfloat16),
            pltpu.VMEM((k, 3 * d), jnp.bfloat16),
            pltpu.VMEM((1, k), jnp.float32),
        ],
        compiler_params=pltpu.CompilerParams(
            dimension_semantics=("arbitrary",),
        ),
    )(z_flat, codebook)

    quant_out = quant.reshape(b, s, d)
    codebook_indices = idx.transpose(0, 2, 1).reshape(b, s, 2)
    q_loss = loss[0, 0] * jnp.float32(2.0 / (n_tok * d))
    return quant_out, codebook_indices, q_loss
